# Initial kernel scaffold; baseline (speedup 1.0000x reference)
import jax
import jax.numpy as jnp
from jax.experimental import pallas as pl


def kernel(inputs, params):
    raise NotImplementedError("write your pallas kernel here")


# speedup vs baseline: 1.0181x; 1.0181x over previous
import functools

import jax
import jax.numpy as jnp
from jax import lax
from jax.experimental import pallas as pl
from jax.experimental.pallas import tpu as pltpu
from jax.experimental.pallas import tpu_sc as plsc


_BR = 448
_BK = 2048


def _vq_body(x_ref, cm_ref, idx_ref, *, nk, bk):
    s = x_ref[...].astype(jnp.float32)
    snorm = jnp.sum(s * s, axis=1, keepdims=True)
    minv = None
    mina = None
    for j in range(nk):
        cm = cm_ref[:, j * bk:(j + 1) * bk]
        cnorm = jnp.sum(cm * cm, axis=0, keepdims=True)
        d = snorm - 2.0 * lax.dot(s, cm) + cnorm
        lmin = jnp.min(d, axis=1, keepdims=True)
        col = lax.broadcasted_iota(jnp.int32, d.shape, 1) + j * bk
        larg = jnp.min(jnp.where(d == lmin, col, jnp.int32(2**31 - 1)),
                       axis=1, keepdims=True)
        if j == 0:
            minv, mina = lmin, larg
        else:
            better = lmin < minv
            mina = jnp.where(better, larg, mina)
            minv = jnp.where(better, lmin, minv)
    idx_ref[...] = mina


def _vq_argmin(x, cm):
    b, hh, w, e = x.shape
    s = x.reshape(-1, e).astype(jnp.bfloat16)
    n = s.shape[0]
    k = cm.shape[1]
    body = functools.partial(_vq_body, nk=k // _BK, bk=_BK)
    idx = pl.pallas_call(
        body,
        grid=(n // _BR,),
        in_specs=[pl.BlockSpec((_BR, e), lambda r: (r, 0)),
                  pl.BlockSpec((e, k), lambda r: (0, 0))],
        out_specs=pl.BlockSpec((_BR, 1), lambda r: (r, 0)),
        out_shape=jax.ShapeDtypeStruct((n, 1), jnp.int32),
    )(s, cm)
    return idx.reshape(b, hh, w)


_SC_CHUNK = 104


def _sc_gather(table, idx):
    v, d = table.shape
    dp = 128
    if d != dp:
        table = jnp.pad(table, ((0, 0), (0, dp - d)))
    n = idx.shape[0]
    info = plsc.get_sparse_core_info()
    nc, ns = info.num_cores, info.num_subcores
    nw = nc * ns
    npad = ((n + nw * _SC_CHUNK - 1) // (nw * _SC_CHUNK)) * (nw * _SC_CHUNK)
    cpw = npad // (nw * _SC_CHUNK)
    idx2 = jnp.zeros((npad // _SC_CHUNK, _SC_CHUNK), jnp.int32)
    idx2 = lax.dynamic_update_slice(
        idx2.reshape(-1), idx, (0,)).reshape(npad // _SC_CHUNK, _SC_CHUNK)

    @functools.partial(
        pl.kernel,
        mesh=plsc.VectorSubcoreMesh(core_axis_name="c", subcore_axis_name="s"),
        out_type=jax.ShapeDtypeStruct((npad // _SC_CHUNK, _SC_CHUNK, dp),
                                      jnp.float32),
        scratch_types=[
            pltpu.VMEM((cpw, _SC_CHUNK), jnp.int32),
            pltpu.VMEM((cpw, _SC_CHUNK, dp), jnp.float32),
            pltpu.SemaphoreType.DMA,
        ],
    )
    def gk(table_hbm, idx_hbm, out_hbm, idx_v, rows_v, sem):
        wid = lax.axis_index("s") * nc + lax.axis_index("c")
        base = wid * cpw
        pltpu.sync_copy(idx_hbm.at[pl.ds(base, cpw)], idx_v)
        copies = [pltpu.async_copy(table_hbm.at[idx_v.at[j]], rows_v.at[j], sem)
                  for j in range(cpw)]
        for c in copies:
            c.wait()
        pltpu.sync_copy(rows_v, out_hbm.at[pl.ds(base, cpw)])

    rows = gk(table, idx2)
    return rows.reshape(npad, dp)[:n, :d]


_DN = ('NHWC', 'HWIO', 'NHWC')


def _conv(x, w, b, stride=1):
    y = lax.conv_general_dilated(x, w, (stride, stride), 'SAME',
                                 dimension_numbers=_DN)
    return y + b


def _convT(x, w, b):
    y = lax.conv_transpose(x, w, (2, 2), 'SAME', dimension_numbers=_DN)
    return y + b


def _resblock(x, p):
    r = jax.nn.relu(x)
    r = _conv(r, p['aw'], p['ab'])
    r = jax.nn.relu(r)
    r = _conv(r, p['bw'], p['bb'])
    return r + x


def _encoder(x, p, stride):
    if stride == 4:
        r = jax.nn.relu(_conv(x, p['c1w'], p['c1b'], 2))
        r = jax.nn.relu(_conv(r, p['c2w'], p['c2b'], 2))
        r = _conv(r, p['c3w'], p['c3b'])
    else:
        r = jax.nn.relu(_conv(x, p['c1w'], p['c1b'], 2))
        r = _conv(r, p['c2w'], p['c2b'])
    for rp in p['res']:
        r = _resblock(r, rp)
    return jax.nn.relu(r)


def _decoder(x, p, strides):
    r = _conv(x, p['cinw'], p['cinb'])
    for rp in p['res']:
        r = _resblock(r, rp)
    r = jax.nn.relu(r)
    if strides == 4:
        r = jax.nn.relu(_convT(r, p['t1w'], p['t1b']))
        r = _convT(r, p['t2w'], p['t2b'])
    else:
        r = _convT(r, p['t1w'], p['t1b'])
    return r


def _quantize_top(x, cm):
    ed = cm.shape[0]
    idx = _vq_argmin(x, cm)
    q = jnp.take(jnp.transpose(cm), idx.reshape(-1), axis=0)
    q = q.reshape(x.shape[:3] + (ed,))
    e_loss = jnp.mean((lax.stop_gradient(q) - x) ** 2)
    q_loss = jnp.mean((q - lax.stop_gradient(x)) ** 2)
    return q, idx, e_loss + 0.25 * q_loss


def _quantize_bot(x, cm):
    ed = cm.shape[0]
    s = x.reshape(-1, ed)
    dist = (jnp.sum(s * s, axis=1, keepdims=True) - 2.0 * (s @ cm)
            + jnp.sum(cm * cm, axis=0, keepdims=True))
    idx = jnp.argmin(dist, axis=1)
    q = _sc_gather(jnp.transpose(cm), idx.astype(jnp.int32))
    q = q.reshape(x.shape[:3] + (ed,))
    e_loss = jnp.mean((lax.stop_gradient(q) - x) ** 2)
    q_loss = jnp.mean((q - lax.stop_gradient(x)) ** 2)
    return q, idx.reshape(x.shape[:3]), e_loss + 0.25 * q_loss


def kernel(inputs, params):
    enc_b = _encoder(inputs, params['be'], 4)
    enc_t = _encoder(enc_b, params['te'], 2)
    zt = _conv(enc_t, params['ctw'], params['ctb'])
    qt, it, lt = _quantize_top(zt, params['cm_t'])
    dt = _decoder(qt, params['dt'], 2)
    cat = jnp.concatenate([dt, enc_b], axis=-1)
    zb = _conv(cat, params['cbw'], params['cbb'])
    qb, ib, lb = _quantize_bot(zb, params['cm_b'])
    up = _convT(qt, params['upw'], params['upb'])
    cat2 = jnp.concatenate([up, qb], axis=-1)
    recon = _decoder(cat2, params['dec'], 4)
    return recon, lt + lb


# speedup vs baseline: 1.2006x; 1.1793x over previous
import functools

import jax
import jax.numpy as jnp
from jax import lax
from jax.experimental import pallas as pl
from jax.experimental.pallas import tpu as pltpu
from jax.experimental.pallas import tpu_sc as plsc


_BR = 448
_BK = 2048


def _vq_body(x_ref, cm_ref, idx_ref, *, nk, bk):
    s = x_ref[...].astype(jnp.float32)
    snorm = jnp.sum(s * s, axis=1, keepdims=True)
    minv = None
    mina = None
    for j in range(nk):
        cm = cm_ref[:, j * bk:(j + 1) * bk]
        cnorm = jnp.sum(cm * cm, axis=0, keepdims=True)
        d = snorm - 2.0 * lax.dot(s, cm) + cnorm
        lmin = jnp.min(d, axis=1, keepdims=True)
        col = lax.broadcasted_iota(jnp.int32, d.shape, 1) + j * bk
        larg = jnp.min(jnp.where(d == lmin, col, jnp.int32(2**31 - 1)),
                       axis=1, keepdims=True)
        if j == 0:
            minv, mina = lmin, larg
        else:
            better = lmin < minv
            mina = jnp.where(better, larg, mina)
            minv = jnp.where(better, lmin, minv)
    idx_ref[...] = mina


def _vq_argmin(x, cm):
    b, hh, w, e = x.shape
    s = x.reshape(-1, e).astype(jnp.bfloat16)
    n = s.shape[0]
    k = cm.shape[1]
    body = functools.partial(_vq_body, nk=k // _BK, bk=_BK)
    idx = pl.pallas_call(
        body,
        grid=(n // _BR,),
        in_specs=[pl.BlockSpec((_BR, e), lambda r: (r, 0)),
                  pl.BlockSpec((e, k), lambda r: (0, 0))],
        out_specs=pl.BlockSpec((_BR, 1), lambda r: (r, 0)),
        out_shape=jax.ShapeDtypeStruct((n, 1), jnp.int32),
    )(s, cm)
    return idx.reshape(b, hh, w)


_SC_CHUNK = 104


def _sc_gather(table, idx):
    v, d = table.shape
    dp = 128
    if d != dp:
        table = jnp.pad(table, ((0, 0), (0, dp - d)))
    n = idx.shape[0]
    info = plsc.get_sparse_core_info()
    nc, ns = info.num_cores, info.num_subcores
    nw = nc * ns
    npad = ((n + nw * _SC_CHUNK - 1) // (nw * _SC_CHUNK)) * (nw * _SC_CHUNK)
    cpw = npad // (nw * _SC_CHUNK)
    idx2 = jnp.zeros((npad // _SC_CHUNK, _SC_CHUNK), jnp.int32)
    idx2 = lax.dynamic_update_slice(
        idx2.reshape(-1), idx, (0,)).reshape(npad // _SC_CHUNK, _SC_CHUNK)

    rows_per_sub = v // ns

    @functools.partial(
        pl.kernel,
        mesh=plsc.VectorSubcoreMesh(core_axis_name="c", subcore_axis_name="s"),
        out_type=jax.ShapeDtypeStruct((npad // _SC_CHUNK, _SC_CHUNK, dp),
                                      jnp.float32),
        scratch_types=[
            pltpu.VMEM((cpw, _SC_CHUNK), jnp.int32),
            pltpu.VMEM((cpw, _SC_CHUNK, dp), jnp.float32),
            pltpu.VMEM_SHARED((v, dp), jnp.float32),
            pltpu.SemaphoreType.DMA,
        ],
    )
    def gk(table_hbm, idx_hbm, out_hbm, idx_v, rows_v, tab_s, sem):
        cid = lax.axis_index("c")
        sid = lax.axis_index("s")
        wid = sid * nc + cid
        base = wid * cpw
        pltpu.sync_copy(table_hbm.at[pl.ds(sid * rows_per_sub, rows_per_sub)],
                        tab_s.at[pl.ds(sid * rows_per_sub, rows_per_sub)])
        pltpu.sync_copy(idx_hbm.at[pl.ds(base, cpw)], idx_v)
        plsc.subcore_barrier()
        copies = [pltpu.async_copy(tab_s.at[idx_v.at[j]], rows_v.at[j], sem)
                  for j in range(cpw)]
        for c in copies:
            c.wait()
        pltpu.sync_copy(rows_v, out_hbm.at[pl.ds(base, cpw)])

    rows = gk(table, idx2)
    return rows.reshape(npad, dp)[:n, :d]


_DN = ('NHWC', 'HWIO', 'NHWC')


def _conv(x, w, b, stride=1):
    y = lax.conv_general_dilated(x, w, (stride, stride), 'SAME',
                                 dimension_numbers=_DN)
    return y + b


def _convT(x, w, b):
    y = lax.conv_transpose(x, w, (2, 2), 'SAME', dimension_numbers=_DN)
    return y + b


def _resblock(x, p):
    r = jax.nn.relu(x)
    r = _conv(r, p['aw'], p['ab'])
    r = jax.nn.relu(r)
    r = _conv(r, p['bw'], p['bb'])
    return r + x


def _encoder(x, p, stride):
    if stride == 4:
        r = jax.nn.relu(_conv(x, p['c1w'], p['c1b'], 2))
        r = jax.nn.relu(_conv(r, p['c2w'], p['c2b'], 2))
        r = _conv(r, p['c3w'], p['c3b'])
    else:
        r = jax.nn.relu(_conv(x, p['c1w'], p['c1b'], 2))
        r = _conv(r, p['c2w'], p['c2b'])
    for rp in p['res']:
        r = _resblock(r, rp)
    return jax.nn.relu(r)


def _decoder(x, p, strides):
    r = _conv(x, p['cinw'], p['cinb'])
    for rp in p['res']:
        r = _resblock(r, rp)
    r = jax.nn.relu(r)
    if strides == 4:
        r = jax.nn.relu(_convT(r, p['t1w'], p['t1b']))
        r = _convT(r, p['t2w'], p['t2b'])
    else:
        r = _convT(r, p['t1w'], p['t1b'])
    return r


def _quantize_top(x, cm):
    ed = cm.shape[0]
    idx = _vq_argmin(x, cm)
    q = jnp.take(jnp.transpose(cm), idx.reshape(-1), axis=0)
    q = q.reshape(x.shape[:3] + (ed,))
    e_loss = jnp.mean((lax.stop_gradient(q) - x) ** 2)
    q_loss = jnp.mean((q - lax.stop_gradient(x)) ** 2)
    return q, idx, e_loss + 0.25 * q_loss


def _quantize_bot(x, cm):
    ed = cm.shape[0]
    s = x.reshape(-1, ed)
    dist = (jnp.sum(s * s, axis=1, keepdims=True) - 2.0 * (s @ cm)
            + jnp.sum(cm * cm, axis=0, keepdims=True))
    idx = jnp.argmin(dist, axis=1)
    q = _sc_gather(jnp.transpose(cm), idx.astype(jnp.int32))
    q = q.reshape(x.shape[:3] + (ed,))
    e_loss = jnp.mean((lax.stop_gradient(q) - x) ** 2)
    q_loss = jnp.mean((q - lax.stop_gradient(x)) ** 2)
    return q, idx.reshape(x.shape[:3]), e_loss + 0.25 * q_loss


def kernel(inputs, params):
    enc_b = _encoder(inputs, params['be'], 4)
    enc_t = _encoder(enc_b, params['te'], 2)
    zt = _conv(enc_t, params['ctw'], params['ctb'])
    qt, it, lt = _quantize_top(zt, params['cm_t'])
    dt = _decoder(qt, params['dt'], 2)
    cat = jnp.concatenate([dt, enc_b], axis=-1)
    zb = _conv(cat, params['cbw'], params['cbb'])
    qb, ib, lb = _quantize_bot(zb, params['cm_b'])
    up = _convT(qt, params['upw'], params['upb'])
    cat2 = jnp.concatenate([up, qb], axis=-1)
    recon = _decoder(cat2, params['dec'], 4)
    return recon, lt + lb


# speedup vs baseline: 1.2864x; 1.0715x over previous
import functools

import jax
import jax.numpy as jnp
from jax import lax
from jax.experimental import pallas as pl
from jax.experimental.pallas import tpu as pltpu
from jax.experimental.pallas import tpu_sc as plsc


_BR = 448
_BK = 2048


def _vq_body(x_ref, cm_ref, idx_ref, *, nk, bk):
    s = x_ref[...].astype(jnp.float32)
    snorm = jnp.sum(s * s, axis=1, keepdims=True)
    minv = None
    mina = None
    for j in range(nk):
        cm = cm_ref[:, j * bk:(j + 1) * bk]
        cnorm = jnp.sum(cm * cm, axis=0, keepdims=True)
        d = snorm - 2.0 * lax.dot(s, cm) + cnorm
        lmin = jnp.min(d, axis=1, keepdims=True)
        col = lax.broadcasted_iota(jnp.int32, d.shape, 1) + j * bk
        larg = jnp.min(jnp.where(d == lmin, col, jnp.int32(2**31 - 1)),
                       axis=1, keepdims=True)
        if j == 0:
            minv, mina = lmin, larg
        else:
            better = lmin < minv
            mina = jnp.where(better, larg, mina)
            minv = jnp.where(better, lmin, minv)
    idx_ref[...] = mina


def _vq_argmin(x, cm):
    b, hh, w, e = x.shape
    s = x.reshape(-1, e).astype(jnp.bfloat16)
    n = s.shape[0]
    k = cm.shape[1]
    body = functools.partial(_vq_body, nk=k // _BK, bk=_BK)
    idx = pl.pallas_call(
        body,
        grid=(n // _BR,),
        in_specs=[pl.BlockSpec((_BR, e), lambda r: (r, 0)),
                  pl.BlockSpec((e, k), lambda r: (0, 0))],
        out_specs=pl.BlockSpec((_BR, 1), lambda r: (r, 0)),
        out_shape=jax.ShapeDtypeStruct((n, 1), jnp.int32),
    )(s, cm)
    return idx.reshape(b, hh, w)


_SC_CHUNK = 104


def _sc_gather(table, idx):
    v, d = table.shape
    dp = 128
    if d != dp:
        table = jnp.pad(table, ((0, 0), (0, dp - d)))
    n = idx.shape[0]
    info = plsc.get_sparse_core_info()
    nc, ns = info.num_cores, info.num_subcores
    nw = nc * ns
    npad = ((n + nw * _SC_CHUNK - 1) // (nw * _SC_CHUNK)) * (nw * _SC_CHUNK)
    cpw = npad // (nw * _SC_CHUNK)
    idx2 = jnp.zeros((npad // _SC_CHUNK, _SC_CHUNK), jnp.int32)
    idx2 = lax.dynamic_update_slice(
        idx2.reshape(-1), idx, (0,)).reshape(npad // _SC_CHUNK, _SC_CHUNK)

    rows_per_sub = v // ns

    @functools.partial(
        pl.kernel,
        mesh=plsc.VectorSubcoreMesh(core_axis_name="c", subcore_axis_name="s"),
        out_type=jax.ShapeDtypeStruct((npad // _SC_CHUNK, _SC_CHUNK, dp),
                                      jnp.float32),
        scratch_types=[
            pltpu.VMEM((cpw, _SC_CHUNK), jnp.int32),
            pltpu.VMEM((cpw, _SC_CHUNK, dp), jnp.float32),
            pltpu.VMEM_SHARED((v, dp), jnp.float32),
            pltpu.SemaphoreType.DMA,
        ],
    )
    def gk(table_hbm, idx_hbm, out_hbm, idx_v, rows_v, tab_s, sem):
        cid = lax.axis_index("c")
        sid = lax.axis_index("s")
        wid = sid * nc + cid
        base = wid * cpw
        pltpu.sync_copy(table_hbm.at[pl.ds(sid * rows_per_sub, rows_per_sub)],
                        tab_s.at[pl.ds(sid * rows_per_sub, rows_per_sub)])
        pltpu.sync_copy(idx_hbm.at[pl.ds(base, cpw)], idx_v)
        plsc.subcore_barrier()
        copies = [pltpu.async_copy(tab_s.at[idx_v.at[j]], rows_v.at[j], sem)
                  for j in range(cpw)]
        for c in copies:
            c.wait()
        pltpu.sync_copy(rows_v, out_hbm.at[pl.ds(base, cpw)])

    rows = gk(table, idx2)
    return rows.reshape(npad, dp)[:n, :d]


_DN = ('NHWC', 'HWIO', 'NHWC')


def _conv(x, w, b, stride=1):
    y = lax.conv_general_dilated(x, w, (stride, stride), 'SAME',
                                 dimension_numbers=_DN)
    return y + b


def _convT(x, w, b):
    y = lax.conv_transpose(x, w, (2, 2), 'SAME', dimension_numbers=_DN)
    return y + b


def _resblock(x, p):
    r = jax.nn.relu(x)
    r = _conv(r, p['aw'], p['ab'])
    r = jax.nn.relu(r)
    r = _conv(r, p['bw'], p['bb'])
    return r + x


def _encoder(x, p, stride):
    if stride == 4:
        r = jax.nn.relu(_conv(x, p['c1w'], p['c1b'], 2))
        r = jax.nn.relu(_conv(r, p['c2w'], p['c2b'], 2))
        r = _conv(r, p['c3w'], p['c3b'])
    else:
        r = jax.nn.relu(_conv(x, p['c1w'], p['c1b'], 2))
        r = _conv(r, p['c2w'], p['c2b'])
    for rp in p['res']:
        r = _resblock(r, rp)
    return jax.nn.relu(r)


def _decoder(x, p, strides):
    r = _conv(x, p['cinw'], p['cinb'])
    for rp in p['res']:
        r = _resblock(r, rp)
    r = jax.nn.relu(r)
    if strides == 4:
        r = jax.nn.relu(_convT(r, p['t1w'], p['t1b']))
        r = _convT(r, p['t2w'], p['t2b'])
    else:
        r = _convT(r, p['t1w'], p['t1b'])
    return r


def _quantize_top(x, cm):
    ed = cm.shape[0]
    idx = _vq_argmin(x, cm)
    q = _sc_gather(jnp.transpose(cm), idx.reshape(-1))
    q = q.reshape(x.shape[:3] + (ed,))
    e_loss = jnp.mean((lax.stop_gradient(q) - x) ** 2)
    q_loss = jnp.mean((q - lax.stop_gradient(x)) ** 2)
    return q, idx, e_loss + 0.25 * q_loss


def _quantize_bot(x, cm):
    ed = cm.shape[0]
    s = x.reshape(-1, ed)
    dist = (jnp.sum(s * s, axis=1, keepdims=True) - 2.0 * (s @ cm)
            + jnp.sum(cm * cm, axis=0, keepdims=True))
    idx = jnp.argmin(dist, axis=1)
    q = _sc_gather(jnp.transpose(cm), idx.astype(jnp.int32))
    q = q.reshape(x.shape[:3] + (ed,))
    e_loss = jnp.mean((lax.stop_gradient(q) - x) ** 2)
    q_loss = jnp.mean((q - lax.stop_gradient(x)) ** 2)
    return q, idx.reshape(x.shape[:3]), e_loss + 0.25 * q_loss


def kernel(inputs, params):
    enc_b = _encoder(inputs, params['be'], 4)
    enc_t = _encoder(enc_b, params['te'], 2)
    zt = _conv(enc_t, params['ctw'], params['ctb'])
    qt, it, lt = _quantize_top(zt, params['cm_t'])
    dt = _decoder(qt, params['dt'], 2)
    cat = jnp.concatenate([dt, enc_b], axis=-1)
    zb = _conv(cat, params['cbw'], params['cbb'])
    qb, ib, lb = _quantize_bot(zb, params['cm_b'])
    up = _convT(qt, params['upw'], params['upb'])
    cat2 = jnp.concatenate([up, qb], axis=-1)
    recon = _decoder(cat2, params['dec'], 4)
    return recon, lt + lb
